# baseline (speedup 1.0000x reference)
import functools
import math

import jax
import jax.numpy as jnp
from jax import lax
from jax.experimental import pallas as pl
from jax.experimental.pallas import tpu as pltpu
from jax.experimental.pallas import tpu_sc as plsc

NUM_FIELDS = 26
FIELD_DIM = 100000
DIM = 16
EPS = 1e-5

NC = 2
NS = 16
NW = NC * NS
LANES = 16

CHUNK = 64
ROWS = CHUNK * NUM_FIELDS
IDX_ROWS = ROWS // 128


def _sc_gather_cross(xi_flat, emb_table, lin_flat, batch):
    per_w = batch // NW
    n_chunks = per_w // CHUNK
    mesh = plsc.VectorSubcoreMesh(core_axis_name="c", subcore_axis_name="s")

    @functools.partial(
        pl.kernel,
        out_type=[
            jax.ShapeDtypeStruct((batch, DIM), jnp.float32),
            jax.ShapeDtypeStruct((batch * NUM_FIELDS,), jnp.float32),
        ],
        mesh=mesh,
        compiler_params=pltpu.CompilerParams(use_tc_tiling_on_sc=False),
        scratch_types=[
            pltpu.VMEM((ROWS,), jnp.int32),
            pltpu.VMEM((ROWS, DIM), jnp.float32),
            pltpu.VMEM((ROWS,), jnp.float32),
            pltpu.VMEM((CHUNK, DIM), jnp.float32),
            pltpu.SemaphoreType.DMA,
            pltpu.SemaphoreType.DMA,
        ],
    )
    def sc_kernel(xi_hbm, emb_hbm, lin_hbm, cross_hbm, linval_hbm,
                  idx_v, rows_v, linv_v, cross_v, sem_e, sem_l):
        wid = lax.axis_index("s") * NC + lax.axis_index("c")

        def chunk_body(c, _):
            base_e = wid * per_w + c * CHUNK
            i0 = base_e * NUM_FIELDS

            pltpu.sync_copy(xi_hbm.at[pl.ds(i0, ROWS)], idx_v)

            copies = []
            for j in range(IDX_ROWS):
                copies.append(pltpu.async_copy(
                    emb_hbm.at[idx_v.at[pl.ds(j * 128, 128)]],
                    rows_v.at[pl.ds(j * 128, 128)], sem_e))
            for j in range(IDX_ROWS):
                copies.append(pltpu.async_copy(
                    lin_hbm.at[idx_v.at[pl.ds(j * 128, 128)]],
                    linv_v.at[pl.ds(j * 128, 128)], sem_l))
            for cp in copies:
                cp.wait()

            zero = jnp.zeros((LANES,), jnp.float32)

            def elem_body(e, _):
                s = zero
                sq = zero
                base = e * NUM_FIELDS
                for f in range(NUM_FIELDS):
                    v = rows_v[base + f]
                    s = s + v
                    sq = sq + v * v
                cross_v[e] = 0.5 * (s * s - sq)
                return 0

            lax.fori_loop(0, CHUNK, elem_body, 0, unroll=False)

            pltpu.sync_copy(cross_v, cross_hbm.at[pl.ds(base_e, CHUNK)])
            pltpu.sync_copy(linv_v, linval_hbm.at[pl.ds(i0, ROWS)])
            return 0

        lax.fori_loop(0, n_chunks, chunk_body, 0, unroll=False)

    return sc_kernel(xi_flat, emb_table, lin_flat)


def _mlp_kernel(cross_ref, linval_ref, bn0g_ref, bn0b_ref, w1_ref, b1_ref,
                bn1g_ref, bn1b_ref, w2_ref, b2_ref, bn2g_ref, bn2b_ref,
                wo_ref, const_ref, out_ref):
    inv = jnp.float32(1.0 / math.sqrt(1.0 + EPS))
    xb = cross_ref[...] * (bn0g_ref[...] * inv) + bn0b_ref[...]
    lin = jnp.sum(linval_ref[...], axis=1, keepdims=True)
    h = jnp.dot(xb, w1_ref[...], preferred_element_type=jnp.float32)
    h = (h + b1_ref[...]) * (bn1g_ref[...] * inv) + bn1b_ref[...]
    h = jnp.maximum(h, 0.0)
    h = jnp.dot(h, w2_ref[...], preferred_element_type=jnp.float32)
    h = (h + b2_ref[...]) * (bn2g_ref[...] * inv) + bn2b_ref[...]
    h = jnp.maximum(h, 0.0)
    o = jnp.dot(h, wo_ref[...], preferred_element_type=jnp.float32)
    out_ref[...] = o + lin + const_ref[...]


def kernel(x, emb_table, lin_table, lin_bias, bn0_g, bn0_b, W1, b1,
           bn1_g, bn1_b, W2, b2, bn2_g, bn2_b, Wo, bo):
    batch = x.shape[0]
    offsets = (jnp.arange(NUM_FIELDS) * FIELD_DIM).astype(jnp.int32)
    xi = (x.astype(jnp.int32) + offsets[None, :]).reshape(-1)

    cross, linval = _sc_gather_cross(xi, emb_table, lin_table.reshape(-1),
                                     batch)

    bk = 2048
    grid = (batch // bk,)
    row = lambda a: a.reshape(1, -1)
    full = lambda shape: pl.BlockSpec(shape, lambda i: (0, 0))
    const = (lin_bias + bo).reshape(1, 1)

    out = pl.pallas_call(
        _mlp_kernel,
        grid=grid,
        in_specs=[
            pl.BlockSpec((bk, DIM), lambda i: (i, 0)),
            pl.BlockSpec((bk, NUM_FIELDS), lambda i: (i, 0)),
            full((1, DIM)), full((1, DIM)),
            full((DIM, 64)), full((1, 64)), full((1, 64)), full((1, 64)),
            full((64, 32)), full((1, 32)), full((1, 32)), full((1, 32)),
            full((32, 1)), full((1, 1)),
        ],
        out_specs=pl.BlockSpec((bk, 1), lambda i: (i, 0)),
        out_shape=jax.ShapeDtypeStruct((batch, 1), jnp.float32),
    )(cross, linval.reshape(batch, NUM_FIELDS), row(bn0_g), row(bn0_b),
      W1, row(b1),
      row(bn1_g), row(bn1_b), W2, row(b2), row(bn2_g), row(bn2_b), Wo, const)

    return out.reshape(batch)


# speedup vs baseline: 2.9593x; 2.9593x over previous
import functools
import math

import jax
import jax.numpy as jnp
from jax import lax
from jax.experimental import pallas as pl
from jax.experimental.pallas import tpu as pltpu
from jax.experimental.pallas import tpu_sc as plsc

NUM_FIELDS = 26
FIELD_DIM = 100000
DIM = 16
EPS = 1e-5

NC = 2
NS = 16
NW = NC * NS
LANES = 16

CHUNK = 64
ROWS = CHUNK * NUM_FIELDS
IDX_ROWS = ROWS // 128


def _sc_gather_cross(xi_flat, xip_flat, emb_table, lin_flat, batch):
    per_w = batch // NW
    n_chunks = per_w // CHUNK
    mesh = plsc.VectorSubcoreMesh(core_axis_name="c", subcore_axis_name="s")

    @functools.partial(
        pl.kernel,
        out_type=[
            jax.ShapeDtypeStruct((batch, DIM), jnp.float32),
            jax.ShapeDtypeStruct((batch * NUM_FIELDS,), jnp.float32),
        ],
        mesh=mesh,
        compiler_params=pltpu.CompilerParams(use_tc_tiling_on_sc=False),
        scratch_types=[
            pltpu.VMEM((ROWS,), jnp.int32),
            pltpu.VMEM((ROWS,), jnp.int32),
            pltpu.VMEM((ROWS, DIM), jnp.float32),
            pltpu.VMEM((ROWS,), jnp.float32),
            pltpu.VMEM((CHUNK, DIM), jnp.float32),
            pltpu.SemaphoreType.DMA,
            pltpu.SemaphoreType.DMA,
        ],
    )
    def sc_kernel(xi_hbm, xip_hbm, emb_hbm, lin_hbm, cross_hbm, linval_hbm,
                  idx_v, idxp_v, rows_v, linv_v, cross_v, sem_e, sem_l):
        wid = lax.axis_index("s") * NC + lax.axis_index("c")

        def chunk_body(c, _):
            base_e = wid * per_w + c * CHUNK
            i0 = base_e * NUM_FIELDS

            pltpu.sync_copy(xi_hbm.at[pl.ds(i0, ROWS)], idx_v)
            pltpu.sync_copy(xip_hbm.at[pl.ds(i0, ROWS)], idxp_v)

            copies = []
            for j in range(IDX_ROWS):
                copies.append(pltpu.async_copy(
                    emb_hbm.at[idxp_v.at[pl.ds(j * 128, 128)]],
                    rows_v.at[pl.ds(j * 128, 128)], sem_e))
            for j in range(IDX_ROWS):
                copies.append(pltpu.async_copy(
                    lin_hbm.at[idx_v.at[pl.ds(j * 128, 128)]],
                    linv_v.at[pl.ds(j * 128, 128)], sem_l))
            for cp in copies:
                cp.wait()

            zero = jnp.zeros((LANES,), jnp.float32)

            def elem_body(e, _):
                s = zero
                sq = zero
                base = e * NUM_FIELDS
                for f in range(NUM_FIELDS):
                    v = rows_v[base + f]
                    s = s + v
                    sq = sq + v * v
                cross_v[e] = 0.5 * (s * s - sq)
                return 0

            lax.fori_loop(0, CHUNK, elem_body, 0, unroll=False)

            pltpu.sync_copy(cross_v, cross_hbm.at[pl.ds(base_e, CHUNK)])
            pltpu.sync_copy(linv_v, linval_hbm.at[pl.ds(i0, ROWS)])
            return 0

        lax.fori_loop(0, n_chunks, chunk_body, 0, unroll=False)

    return sc_kernel(xi_flat, xip_flat, emb_table, lin_flat)


def _transpose_kernel(src_ref, dst_ref):
    blk = src_ref.shape[1]
    for k in range(blk // 1024):
        x8 = jnp.concatenate(
            [src_ref[:, k * 1024 + j * 128: k * 1024 + (j + 1) * 128]
             for j in range(8)], axis=0)
        dst_ref[k * 128:(k + 1) * 128, :] = x8.T


def _to_row_major(emb_t, blk):
    total = emb_t.shape[1]
    grid = (pl.cdiv(total, blk),)
    br = blk * DIM // 128
    flat = pl.pallas_call(
        _transpose_kernel,
        grid=grid,
        in_specs=[pl.BlockSpec((DIM, blk), lambda i: (0, i))],
        out_specs=pl.BlockSpec((br, 128), lambda i: (i, 0)),
        out_shape=jax.ShapeDtypeStruct((total * DIM // 128, 128), jnp.float32),
    )(emb_t)
    return flat.reshape(total, DIM)


def _mlp_kernel(cross_ref, linval_ref, bn0g_ref, bn0b_ref, w1_ref, b1_ref,
                bn1g_ref, bn1b_ref, w2_ref, b2_ref, bn2g_ref, bn2b_ref,
                wo_ref, const_ref, out_ref):
    inv = jnp.float32(1.0 / math.sqrt(1.0 + EPS))
    xb = cross_ref[...] * (bn0g_ref[...] * inv) + bn0b_ref[...]
    lin = jnp.sum(linval_ref[...], axis=1, keepdims=True)
    h = jnp.dot(xb, w1_ref[...], preferred_element_type=jnp.float32)
    h = (h + b1_ref[...]) * (bn1g_ref[...] * inv) + bn1b_ref[...]
    h = jnp.maximum(h, 0.0)
    h = jnp.dot(h, w2_ref[...], preferred_element_type=jnp.float32)
    h = (h + b2_ref[...]) * (bn2g_ref[...] * inv) + bn2b_ref[...]
    h = jnp.maximum(h, 0.0)
    o = jnp.dot(h, wo_ref[...], preferred_element_type=jnp.float32)
    out_ref[...] = o + lin + const_ref[...]


def kernel(x, emb_table, lin_table, lin_bias, bn0_g, bn0_b, W1, b1,
           bn1_g, bn1_b, W2, b2, bn2_g, bn2_b, Wo, bo):
    batch = x.shape[0]
    offsets = (jnp.arange(NUM_FIELDS) * FIELD_DIM).astype(jnp.int32)
    xi = (x.astype(jnp.int32) + offsets[None, :]).reshape(-1)
    xip = ((xi // 1024) * 1024 + (xi % 128) * 8 + (xi // 128) % 8)

    emb_rm = _to_row_major(emb_table.T, 16384)
    cross, linval = _sc_gather_cross(xi, xip, emb_rm,
                                     jnp.squeeze(lin_table, 1), batch)

    bk = 2048
    grid = (batch // bk,)
    row = lambda a: a.reshape(1, -1)
    full = lambda shape: pl.BlockSpec(shape, lambda i: (0, 0))
    const = (lin_bias + bo).reshape(1, 1)

    out = pl.pallas_call(
        _mlp_kernel,
        grid=grid,
        in_specs=[
            pl.BlockSpec((bk, DIM), lambda i: (i, 0)),
            pl.BlockSpec((bk, NUM_FIELDS), lambda i: (i, 0)),
            full((1, DIM)), full((1, DIM)),
            full((DIM, 64)), full((1, 64)), full((1, 64)), full((1, 64)),
            full((64, 32)), full((1, 32)), full((1, 32)), full((1, 32)),
            full((32, 1)), full((1, 1)),
        ],
        out_specs=pl.BlockSpec((bk, 1), lambda i: (i, 0)),
        out_shape=jax.ShapeDtypeStruct((batch, 1), jnp.float32),
    )(cross, linval.reshape(batch, NUM_FIELDS), row(bn0_g), row(bn0_b),
      W1, row(b1),
      row(bn1_g), row(bn1_b), W2, row(b2), row(bn2_g), row(bn2_b), Wo, const)

    return out.reshape(batch)


# speedup vs baseline: 4.0036x; 1.3529x over previous
import functools
import math

import jax
import jax.numpy as jnp
from jax import lax
from jax.experimental import pallas as pl
from jax.experimental.pallas import tpu as pltpu
from jax.experimental.pallas import tpu_sc as plsc

NUM_FIELDS = 26
FIELD_DIM = 100000
DIM = 16
EPS = 1e-5

NC = 2
NS = 16
NW = NC * NS
LANES = 16

CHUNK = 64
ROWS = CHUNK * NUM_FIELDS
IDX_ROWS = ROWS // 128


def _sc_gather_cross(xi_flat, xip_flat, emb_table, lin_flat, batch):
    per_w = batch // NW
    n_chunks = per_w // CHUNK
    mesh = plsc.VectorSubcoreMesh(core_axis_name="c", subcore_axis_name="s")

    @functools.partial(
        pl.kernel,
        out_type=[
            jax.ShapeDtypeStruct((batch, DIM), jnp.float32),
            jax.ShapeDtypeStruct((batch * NUM_FIELDS,), jnp.float32),
        ],
        mesh=mesh,
        compiler_params=pltpu.CompilerParams(use_tc_tiling_on_sc=False),
        scratch_types=[
            pltpu.VMEM((ROWS,), jnp.int32),
            pltpu.VMEM((ROWS,), jnp.int32),
            pltpu.VMEM((ROWS, DIM), jnp.float32),
            pltpu.VMEM((ROWS,), jnp.float32),
            pltpu.VMEM((CHUNK, DIM), jnp.float32),
            pltpu.SemaphoreType.DMA,
            pltpu.SemaphoreType.DMA,
        ],
    )
    def sc_kernel(xi_hbm, xip_hbm, emb_hbm, lin_hbm, cross_hbm, linval_hbm,
                  idx_v, idxp_v, rows_v, linv_v, cross_v, sem_e, sem_l):
        wid = lax.axis_index("s") * NC + lax.axis_index("c")

        def chunk_body(c, _):
            base_e = wid * per_w + c * CHUNK
            i0 = base_e * NUM_FIELDS

            pltpu.sync_copy(xi_hbm.at[pl.ds(i0, ROWS)], idx_v)
            pltpu.sync_copy(xip_hbm.at[pl.ds(i0, ROWS)], idxp_v)

            copies = []
            for j in range(IDX_ROWS):
                copies.append(pltpu.async_copy(
                    emb_hbm.at[idxp_v.at[pl.ds(j * 128, 128)]],
                    rows_v.at[pl.ds(j * 128, 128)], sem_e))
            for j in range(IDX_ROWS):
                copies.append(pltpu.async_copy(
                    lin_hbm.at[idx_v.at[pl.ds(j * 128, 128)]],
                    linv_v.at[pl.ds(j * 128, 128)], sem_l))
            for cp in copies:
                cp.wait()

            zero = jnp.zeros((LANES,), jnp.float32)

            def elem_body(e, _):
                s = zero
                sq = zero
                base = e * NUM_FIELDS
                for f in range(NUM_FIELDS):
                    v = rows_v[base + f]
                    s = s + v
                    sq = sq + v * v
                cross_v[e] = 0.5 * (s * s - sq)
                return 0

            lax.fori_loop(0, CHUNK, elem_body, 0, unroll=False)

            pltpu.sync_copy(cross_v, cross_hbm.at[pl.ds(base_e, CHUNK)])
            pltpu.sync_copy(linv_v, linval_hbm.at[pl.ds(i0, ROWS)])
            return 0

        lax.fori_loop(0, n_chunks, chunk_body, 0, unroll=False)

    return sc_kernel(xi_flat, xip_flat, emb_table, lin_flat)


def _transpose_kernel(src_ref, lin_ref, dst_ref, lin_out_ref):
    blk = src_ref.shape[1]
    for k in range(blk // 1024):
        x8 = jnp.concatenate(
            [src_ref[:, k * 1024 + j * 128: k * 1024 + (j + 1) * 128]
             for j in range(8)], axis=0)
        dst_ref[k * 128:(k + 1) * 128, :] = x8.T
    for p in range(blk // 1024):
        piece = jnp.concatenate(
            [lin_ref[:, p * 1024 + s * 128: p * 1024 + (s + 1) * 128]
             for s in range(8)], axis=0)
        lin_out_ref[p * 8:(p + 1) * 8, :] = piece


def _to_row_major(emb_t, lin_t, blk):
    total = emb_t.shape[1]
    grid = (pl.cdiv(total, blk),)
    br = blk * DIM // 128
    nrow = pl.cdiv(total, blk) * (blk // 128)
    flat, lin_flat = pl.pallas_call(
        _transpose_kernel,
        grid=grid,
        in_specs=[pl.BlockSpec((DIM, blk), lambda i: (0, i)),
                  pl.BlockSpec((1, blk), lambda i: (0, i))],
        out_specs=[pl.BlockSpec((br, 128), lambda i: (i, 0)),
                   pl.BlockSpec((blk // 128, 128), lambda i: (i, 0))],
        out_shape=[
            jax.ShapeDtypeStruct((total * DIM // 128, 128), jnp.float32),
            jax.ShapeDtypeStruct((nrow, 128), jnp.float32),
        ],
    )(emb_t, lin_t)
    return flat.reshape(total, DIM), lin_flat.reshape(nrow * 128)


def _mlp_kernel(cross_ref, linval_ref, bn0g_ref, bn0b_ref, w1_ref, b1_ref,
                bn1g_ref, bn1b_ref, w2_ref, b2_ref, bn2g_ref, bn2b_ref,
                wo_ref, const_ref, out_ref):
    inv = jnp.float32(1.0 / math.sqrt(1.0 + EPS))
    xb = cross_ref[...] * (bn0g_ref[...] * inv) + bn0b_ref[...]
    lin = jnp.sum(linval_ref[...], axis=1, keepdims=True)
    h = jnp.dot(xb, w1_ref[...], preferred_element_type=jnp.float32)
    h = (h + b1_ref[...]) * (bn1g_ref[...] * inv) + bn1b_ref[...]
    h = jnp.maximum(h, 0.0)
    h = jnp.dot(h, w2_ref[...], preferred_element_type=jnp.float32)
    h = (h + b2_ref[...]) * (bn2g_ref[...] * inv) + bn2b_ref[...]
    h = jnp.maximum(h, 0.0)
    o = jnp.dot(h, wo_ref[...], preferred_element_type=jnp.float32)
    out_ref[...] = o + lin + const_ref[...]


def kernel(x, emb_table, lin_table, lin_bias, bn0_g, bn0_b, W1, b1,
           bn1_g, bn1_b, W2, b2, bn2_g, bn2_b, Wo, bo):
    batch = x.shape[0]
    offsets = (jnp.arange(NUM_FIELDS) * FIELD_DIM).astype(jnp.int32)
    xi = (x.astype(jnp.int32) + offsets[None, :]).reshape(-1)
    xip = ((xi // 1024) * 1024 + (xi % 128) * 8 + (xi // 128) % 8)

    emb_rm, lin_flat = _to_row_major(emb_table.T, lin_table.T, 16384)
    cross, linval = _sc_gather_cross(xi, xip, emb_rm, lin_flat, batch)

    bk = 2048
    grid = (batch // bk,)
    row = lambda a: a.reshape(1, -1)
    full = lambda shape: pl.BlockSpec(shape, lambda i: (0, 0))
    const = (lin_bias + bo).reshape(1, 1)

    out = pl.pallas_call(
        _mlp_kernel,
        grid=grid,
        in_specs=[
            pl.BlockSpec((bk, DIM), lambda i: (i, 0)),
            pl.BlockSpec((bk, NUM_FIELDS), lambda i: (i, 0)),
            full((1, DIM)), full((1, DIM)),
            full((DIM, 64)), full((1, 64)), full((1, 64)), full((1, 64)),
            full((64, 32)), full((1, 32)), full((1, 32)), full((1, 32)),
            full((32, 1)), full((1, 1)),
        ],
        out_specs=pl.BlockSpec((bk, 1), lambda i: (i, 0)),
        out_shape=jax.ShapeDtypeStruct((batch, 1), jnp.float32),
    )(cross, linval.reshape(batch, NUM_FIELDS), row(bn0_g), row(bn0_b),
      W1, row(b1),
      row(bn1_g), row(bn1_b), W2, row(b2), row(bn2_g), row(bn2_b), Wo, const)

    return out.reshape(batch)
